# baseline (speedup 1.0000x reference)
import functools

import jax
import jax.numpy as jnp
from jax import lax
from jax.experimental import pallas as pl
from jax.experimental.pallas import tpu as pltpu
from jax.experimental.pallas import tpu_sc as plsc

B, N_OBJ, Dm = 4, 2048, 768
E = 8
H = 768
O = 768
T = B * N_OBJ
PAIRS = 2 * T
TB = 512
NB = T // TB
TBg = 256
CAP = PAIRS + E * TBg
MAX_TILES = CAP // TBg

NW = 32
CHUNK = PAIRS // NW
SUB = 64
TCHUNK = T // NW
STOK = 32


def _router_body(x_ref, Wr_ref, br_ref, ints_ref, flts_ref, base_ref, cumt_ref,
                 xp_ref, run_ref):
    pid = pl.program_id(0)

    @pl.when(pid == 0)
    def _():
        run_ref[...] = jnp.zeros((1, E), jnp.float32)

    xb = x_ref[...]
    au = lax.bitcast_convert_type(
        xb[:, :Dm // 2].astype(jnp.bfloat16), jnp.uint16).astype(jnp.uint32)
    bu = lax.bitcast_convert_type(
        xb[:, Dm // 2:].astype(jnp.bfloat16), jnp.uint16).astype(jnp.uint32)
    xp_ref[...] = lax.bitcast_convert_type(au | (bu << 16), jnp.float32)
    logits = lax.dot_general(xb, Wr_ref[...], (((1,), (1,)), ((), ())),
                             preferred_element_type=jnp.float32) + br_ref[...]
    ids = lax.broadcasted_iota(jnp.int32, (TB, E), 1)
    m0 = jnp.max(logits, axis=1, keepdims=True)
    a0 = jnp.min(jnp.where(logits == m0, ids, E), axis=1, keepdims=True)
    l1 = jnp.where(ids == a0, -jnp.inf, logits)
    m1 = jnp.max(l1, axis=1, keepdims=True)
    a1 = jnp.min(jnp.where(l1 == m1, ids, E), axis=1, keepdims=True)
    c0 = 1.0 / (1.0 + jnp.exp(m1 - m0))
    c1 = 1.0 - c0

    oh0 = (ids == a0).astype(jnp.float32)
    oh1 = (ids == a1).astype(jnp.float32)
    Hh = oh0 + oh1
    ri = lax.broadcasted_iota(jnp.int32, (TB, TB), 0)
    ci = lax.broadcasted_iota(jnp.int32, (TB, TB), 1)
    Ltri = (ci < ri).astype(jnp.float32)
    run = run_ref[...]
    C = lax.dot_general(Ltri, Hh, (((1,), (0,)), ((), ())),
                        preferred_element_type=jnp.float32) + run
    r0 = jnp.sum(C * oh0, axis=1, keepdims=True)
    r1 = jnp.sum(C * oh1, axis=1, keepdims=True)

    ints_ref[...] = jnp.concatenate(
        [a0, a1, r0.astype(jnp.int32), r1.astype(jnp.int32)], axis=1)
    flts_ref[...] = jnp.concatenate([c0, c1], axis=1)

    newrun = run + jnp.sum(Hh, axis=0, keepdims=True)
    run_ref[...] = newrun

    @pl.when(pid == NB - 1)
    def _():
        n = jnp.floor((newrun + (TBg - 1)) / TBg)
        p = n * TBg
        el = lax.broadcasted_iota(jnp.int32, (E, E), 0)
        ec = lax.broadcasted_iota(jnp.int32, (E, E), 1)
        Mstrict = (el < ec).astype(jnp.float32)
        Mle = (el <= ec).astype(jnp.float32)
        base_ref[...] = lax.dot_general(
            p, Mstrict, (((1,), (0,)), ((), ())),
            preferred_element_type=jnp.float32).astype(jnp.int32)
        cumt_ref[...] = lax.dot_general(
            n, Mle, (((1,), (0,)), ((), ())),
            preferred_element_type=jnp.float32).astype(jnp.int32)


def _router(xf, Wr, brr):
    return pl.pallas_call(
        _router_body,
        grid=(NB,),
        in_specs=[
            pl.BlockSpec((TB, Dm), lambda i: (i, 0)),
            pl.BlockSpec((E, Dm), lambda i: (0, 0)),
            pl.BlockSpec((1, E), lambda i: (0, 0)),
        ],
        out_specs=[
            pl.BlockSpec((TB, 4), lambda i: (i, 0)),
            pl.BlockSpec((TB, 2), lambda i: (i, 0)),
            pl.BlockSpec((1, E), lambda i: (0, 0)),
            pl.BlockSpec((1, E), lambda i: (0, 0)),
            pl.BlockSpec((TB, Dm // 2), lambda i: (i, 0)),
        ],
        out_shape=[
            jax.ShapeDtypeStruct((T, 4), jnp.int32),
            jax.ShapeDtypeStruct((T, 2), jnp.float32),
            jax.ShapeDtypeStruct((1, E), jnp.int32),
            jax.ShapeDtypeStruct((1, E), jnp.int32),
            jax.ShapeDtypeStruct((T, Dm // 2), jnp.float32),
        ],
        scratch_shapes=[pltpu.VMEM((1, E), jnp.float32)],
    )(xf, Wr, brr)


NSUB = CHUNK // SUB


def _dispatch_body(x_hbm, pdest_hbm, eids_hbm, base_hbm, xs_hbm,
                   dest_hbm, base_v, pd_v, e_v, d_v, de_v, do_v,
                   rows_v, seme, semo):
    wid = lax.axis_index("s") * 2 + lax.axis_index("c")
    p0 = pl.multiple_of(wid * CHUNK, CHUNK)
    pltpu.sync_copy(base_hbm, base_v)
    pltpu.sync_copy(pdest_hbm.at[pl.ds(p0, CHUNK)], pd_v)
    pltpu.sync_copy(eids_hbm.at[pl.ds(p0, CHUNK)], e_v)
    for c in range(CHUNK // 16):
        sl = pl.ds(c * 16, 16)
        d_v[sl] = pd_v[sl] + plsc.load_gather(base_v, [e_v[sl]])
    pltpu.sync_copy(d_v, dest_hbm.at[pl.ds(p0, CHUNK)])
    descs = {}
    for j in range(NSUB):
        b = j % 2
        if j >= 2:
            for cp in descs[j - 2]:
                cp.wait()
        t0 = pl.multiple_of(wid * (CHUNK // 2) + j * (SUB // 2), SUB // 2)
        pltpu.sync_copy(x_hbm.at[pl.ds(t0, SUB // 2)], rows_v[b])
        for c in range(SUB // 32):
            sl = pl.ds(c * 16, 16)
            ii = lax.iota(jnp.int32, 16) * 2 + j * SUB + c * 32
            de_v[b][sl] = plsc.load_gather(d_v, [ii])
            do_v[b][sl] = plsc.load_gather(d_v, [ii + 1])
        cp1 = pltpu.async_copy(rows_v[b], xs_hbm.at[de_v[b]], seme[b])
        cp2 = pltpu.async_copy(rows_v[b], xs_hbm.at[do_v[b]], semo[b])
        descs[j] = (cp1, cp2)
    for j in (NSUB - 2, NSUB - 1):
        for cp in descs[j]:
            cp.wait()


_dispatch = functools.partial(
    pl.kernel,
    _dispatch_body,
    compiler_params=pltpu.CompilerParams(needs_layout_passes=False),
    out_type=[
        jax.ShapeDtypeStruct((CAP, Dm // 2), jnp.float32),
        jax.ShapeDtypeStruct((PAIRS,), jnp.int32),
    ],
    scratch_types=[
        pltpu.VMEM((16,), jnp.int32),
        pltpu.VMEM((CHUNK,), jnp.int32),
        pltpu.VMEM((CHUNK,), jnp.int32),
        pltpu.VMEM((CHUNK,), jnp.int32),
        [pltpu.VMEM((SUB // 2,), jnp.int32)] * 2,
        [pltpu.VMEM((SUB // 2,), jnp.int32)] * 2,
        [pltpu.VMEM((SUB // 2, Dm // 2), jnp.float32)] * 2,
        [pltpu.SemaphoreType.DMA] * 2,
        [pltpu.SemaphoreType.DMA] * 2,
    ],
)


def _emap(j, cum):
    e = jnp.int32(0)
    for k in range(E):
        e = e + (j >= cum[k]).astype(jnp.int32)
    return jnp.minimum(e, E - 1)


def _gemm_body(cum_ref, xs_ref, w1a_ref, w1b_ref, b1_ref, w2_ref, b2_ref,
               ys_ref):
    j = pl.program_id(0)

    @pl.when(j < cum_ref[E - 1])
    def _():
        pu = lax.bitcast_convert_type(xs_ref[...], jnp.uint32)
        xa = lax.bitcast_convert_type((pu & 0xFFFF).astype(jnp.uint16),
                                      jnp.bfloat16)
        xb2 = lax.bitcast_convert_type((pu >> 16).astype(jnp.uint16),
                                       jnp.bfloat16)
        h = (lax.dot_general(xa, w1a_ref[0], (((1,), (1,)), ((), ())),
                             preferred_element_type=jnp.float32)
             + lax.dot_general(xb2, w1b_ref[0], (((1,), (1,)), ((), ())),
                               preferred_element_type=jnp.float32)
             + b1_ref[0])
        h = jnp.maximum(h, 0.0).astype(jnp.bfloat16)
        y = lax.dot_general(h, w2_ref[0], (((1,), (1,)), ((), ())),
                            preferred_element_type=jnp.float32) + b2_ref[0]
        yau = lax.bitcast_convert_type(
            y[:, :O // 2].astype(jnp.bfloat16), jnp.uint16).astype(jnp.uint32)
        ybu = lax.bitcast_convert_type(
            y[:, O // 2:].astype(jnp.bfloat16), jnp.uint16).astype(jnp.uint32)
        ys_ref[...] = lax.bitcast_convert_type(yau | (ybu << 16), jnp.float32)


def _gemm(cumt, xs, W1a, W1bh, b1r, W2b, b2r):
    grid_spec = pltpu.PrefetchScalarGridSpec(
        num_scalar_prefetch=1,
        grid=(MAX_TILES,),
        in_specs=[
            pl.BlockSpec((TBg, Dm // 2), lambda j, cum: (j, 0)),
            pl.BlockSpec((1, H, Dm // 2), lambda j, cum: (_emap(j, cum), 0, 0)),
            pl.BlockSpec((1, H, Dm // 2), lambda j, cum: (_emap(j, cum), 0, 0)),
            pl.BlockSpec((1, 1, H), lambda j, cum: (_emap(j, cum), 0, 0)),
            pl.BlockSpec((1, O, H), lambda j, cum: (_emap(j, cum), 0, 0)),
            pl.BlockSpec((1, 1, O), lambda j, cum: (_emap(j, cum), 0, 0)),
        ],
        out_specs=pl.BlockSpec((TBg, O // 2), lambda j, cum: (j, 0)),
    )
    return pl.pallas_call(
        _gemm_body,
        grid_spec=grid_spec,
        out_shape=jax.ShapeDtypeStruct((CAP, O // 2), jnp.float32),
    )(cumt, xs, W1a, W1bh, b1r, W2b, b2r)


NSUBC = TCHUNK // STOK


def _combine_body(ys_hbm, dest_hbm, gates_hbm, out_hbm, d_all, g_all, de_v,
                  do_v, o_v, r_v, oo_v, sg, sr):
    wid = lax.axis_index("s") * 2 + lax.axis_index("c")
    q0 = pl.multiple_of(wid * 2 * TCHUNK, 2 * TCHUNK)
    pltpu.sync_copy(dest_hbm.at[pl.ds(q0, 2 * TCHUNK)], d_all)

    def stage(j, b):
        for c in range(STOK // 16):
            sl = pl.ds(c * 16, 16)
            ii = lax.iota(jnp.int32, 16) * 2 + j * 2 * STOK + c * 32
            de_v[b][sl] = plsc.load_gather(d_all, [ii])
            do_v[b][sl] = plsc.load_gather(d_all, [ii + 1])
        pltpu.sync_copy(
            gates_hbm.at[pl.ds(q0 + j * 2 * STOK, 2 * STOK)], g_all[b])
        ge = pltpu.async_copy(ys_hbm.at[de_v[b]], o_v[b], sg[b])
        go = pltpu.async_copy(ys_hbm.at[do_v[b]], r_v[b], sr[b])
        return ge, go

    gd = {}
    gd[0] = stage(0, 0)
    for j in range(NSUBC):
        b = j % 2
        if j + 1 < NSUBC:
            gd[j + 1] = stage(j + 1, 1 - b)
        gd[j][0].wait()
        gd[j][1].wait()

        def body(t, carry):
            g0 = g_all[b][2 * t, :]
            g1 = g_all[b][2 * t + 1, :]
            for cc in range(Dm // 32):
                sl = pl.ds(cc * 16, 16)
                pe = plsc.bitcast(o_v[b][t, sl], jnp.bfloat16)
                po = plsc.bitcast(r_v[b][t, sl], jnp.bfloat16)
                ae, be = plsc.unpack(pe, format=plsc.PackFormat.INTERLEAVED)
                ao, bo = plsc.unpack(po, format=plsc.PackFormat.INTERLEAVED)
                oo_v[t, pl.ds(cc * 16, 16)] = g0 * ae + g1 * ao
                oo_v[t, pl.ds(Dm // 2 + cc * 16, 16)] = g0 * be + g1 * bo
            return carry

        lax.fori_loop(0, STOK, body, 0)
        t0 = pl.multiple_of(wid * TCHUNK + j * STOK, STOK)
        pltpu.sync_copy(oo_v, out_hbm.at[pl.ds(t0, STOK)])


_combine = functools.partial(
    pl.kernel,
    _combine_body,
    compiler_params=pltpu.CompilerParams(needs_layout_passes=False),
    out_type=jax.ShapeDtypeStruct((T, Dm), jnp.float32),
    scratch_types=[
        pltpu.VMEM((2 * TCHUNK,), jnp.int32),
        [pltpu.VMEM((2 * STOK, 16), jnp.float32)] * 2,
        [pltpu.VMEM((STOK,), jnp.int32)] * 2,
        [pltpu.VMEM((STOK,), jnp.int32)] * 2,
        [pltpu.VMEM((STOK, Dm // 2), jnp.float32)] * 2,
        [pltpu.VMEM((STOK, Dm // 2), jnp.float32)] * 2,
        pltpu.VMEM((STOK, Dm), jnp.float32),
        [pltpu.SemaphoreType.DMA] * 2,
        [pltpu.SemaphoreType.DMA] * 2,
    ],
)


def kernel(x, W1, b1, W2, b2, Wr, br):
    xf = x.reshape(T, Dm)
    ints, flts, base_o, cumt_o, xp = _router(xf, Wr, br.reshape(1, E))
    eids = ints[:, 0:2].reshape(PAIRS)
    pdest = ints[:, 2:4].reshape(PAIRS)
    gates16 = jnp.broadcast_to(flts.reshape(PAIRS, 1), (PAIRS, 16))
    base16 = jnp.concatenate(
        [base_o.reshape(E), jnp.zeros((16 - E,), jnp.int32)])
    cumt = cumt_o.reshape(E)

    mesh = plsc.VectorSubcoreMesh(core_axis_name="c", subcore_axis_name="s")
    xs, dest = _dispatch(mesh=mesh)(xp, pdest, eids, base16)
    W1bf = W1.astype(jnp.bfloat16)
    ys = _gemm(cumt, xs, W1bf[:, :, :Dm // 2], W1bf[:, :, Dm // 2:],
               b1.reshape(E, 1, H), W2.astype(jnp.bfloat16),
               b2.reshape(E, 1, O))
    out = _combine(mesh=mesh)(ys, dest, gates16)
    return out.reshape(B, N_OBJ, O)


# speedup vs baseline: 1.0710x; 1.0710x over previous
import functools

import jax
import jax.numpy as jnp
from jax import lax
from jax.experimental import pallas as pl
from jax.experimental.pallas import tpu as pltpu
from jax.experimental.pallas import tpu_sc as plsc

B, N_OBJ, Dm = 4, 2048, 768
E = 8
H = 768
O = 768
T = B * N_OBJ
PAIRS = 2 * T
TB = 512
NB = T // TB
TBg = 256
CAP = PAIRS + E * TBg
MAX_TILES = CAP // TBg

NW = 32
CHUNK = PAIRS // NW
SUB = 64
TCHUNK = T // NW
STOK = 16


def _router_body(x_ref, Wr_ref, br_ref, ints_ref, flts_ref, base_ref, cumt_ref,
                 xp_ref, run_ref):
    pid = pl.program_id(0)

    @pl.when(pid == 0)
    def _():
        run_ref[...] = jnp.zeros((1, E), jnp.float32)

    xb = x_ref[...]
    au = lax.bitcast_convert_type(
        xb[:, :Dm // 2].astype(jnp.bfloat16), jnp.uint16).astype(jnp.uint32)
    bu = lax.bitcast_convert_type(
        xb[:, Dm // 2:].astype(jnp.bfloat16), jnp.uint16).astype(jnp.uint32)
    xp_ref[...] = lax.bitcast_convert_type(au | (bu << 16), jnp.float32)
    logits = lax.dot_general(xb, Wr_ref[...], (((1,), (1,)), ((), ())),
                             preferred_element_type=jnp.float32) + br_ref[...]
    ids = lax.broadcasted_iota(jnp.int32, (TB, E), 1)
    m0 = jnp.max(logits, axis=1, keepdims=True)
    a0 = jnp.min(jnp.where(logits == m0, ids, E), axis=1, keepdims=True)
    l1 = jnp.where(ids == a0, -jnp.inf, logits)
    m1 = jnp.max(l1, axis=1, keepdims=True)
    a1 = jnp.min(jnp.where(l1 == m1, ids, E), axis=1, keepdims=True)
    c0 = 1.0 / (1.0 + jnp.exp(m1 - m0))
    c1 = 1.0 - c0

    oh0 = (ids == a0).astype(jnp.float32)
    oh1 = (ids == a1).astype(jnp.float32)
    Hh = oh0 + oh1
    ri = lax.broadcasted_iota(jnp.int32, (TB, TB), 0)
    ci = lax.broadcasted_iota(jnp.int32, (TB, TB), 1)
    Ltri = (ci < ri).astype(jnp.float32)
    run = run_ref[...]
    C = lax.dot_general(Ltri, Hh, (((1,), (0,)), ((), ())),
                        preferred_element_type=jnp.float32) + run
    r0 = jnp.sum(C * oh0, axis=1, keepdims=True)
    r1 = jnp.sum(C * oh1, axis=1, keepdims=True)

    ints_ref[...] = jnp.concatenate(
        [a0, a1, r0.astype(jnp.int32), r1.astype(jnp.int32)], axis=1)
    flts_ref[...] = jnp.concatenate([c0, c1], axis=1)

    newrun = run + jnp.sum(Hh, axis=0, keepdims=True)
    run_ref[...] = newrun

    @pl.when(pid == NB - 1)
    def _():
        n = jnp.floor((newrun + (TBg - 1)) / TBg)
        p = n * TBg
        el = lax.broadcasted_iota(jnp.int32, (E, E), 0)
        ec = lax.broadcasted_iota(jnp.int32, (E, E), 1)
        Mstrict = (el < ec).astype(jnp.float32)
        Mle = (el <= ec).astype(jnp.float32)
        base_ref[...] = lax.dot_general(
            p, Mstrict, (((1,), (0,)), ((), ())),
            preferred_element_type=jnp.float32).astype(jnp.int32)
        cumt_ref[...] = lax.dot_general(
            n, Mle, (((1,), (0,)), ((), ())),
            preferred_element_type=jnp.float32).astype(jnp.int32)


def _router(xf, Wr, brr):
    return pl.pallas_call(
        _router_body,
        grid=(NB,),
        in_specs=[
            pl.BlockSpec((TB, Dm), lambda i: (i, 0)),
            pl.BlockSpec((E, Dm), lambda i: (0, 0)),
            pl.BlockSpec((1, E), lambda i: (0, 0)),
        ],
        out_specs=[
            pl.BlockSpec((TB, 4), lambda i: (i, 0)),
            pl.BlockSpec((TB, 2), lambda i: (i, 0)),
            pl.BlockSpec((1, E), lambda i: (0, 0)),
            pl.BlockSpec((1, E), lambda i: (0, 0)),
            pl.BlockSpec((TB, Dm // 2), lambda i: (i, 0)),
        ],
        out_shape=[
            jax.ShapeDtypeStruct((T, 4), jnp.int32),
            jax.ShapeDtypeStruct((T, 2), jnp.float32),
            jax.ShapeDtypeStruct((1, E), jnp.int32),
            jax.ShapeDtypeStruct((1, E), jnp.int32),
            jax.ShapeDtypeStruct((T, Dm // 2), jnp.float32),
        ],
        scratch_shapes=[pltpu.VMEM((1, E), jnp.float32)],
    )(xf, Wr, brr)


NSUB = CHUNK // SUB


def _dispatch_body(x_hbm, pdest_hbm, eids_hbm, base_hbm, xs_hbm,
                   dest_hbm, base_v, pd_v, e_v, d_v, de_v, do_v,
                   rows_v, seme, semo):
    wid = lax.axis_index("s") * 2 + lax.axis_index("c")
    p0 = pl.multiple_of(wid * CHUNK, CHUNK)
    pltpu.sync_copy(base_hbm, base_v)
    pltpu.sync_copy(pdest_hbm.at[pl.ds(p0, CHUNK)], pd_v)
    pltpu.sync_copy(eids_hbm.at[pl.ds(p0, CHUNK)], e_v)
    for c in range(CHUNK // 16):
        sl = pl.ds(c * 16, 16)
        d_v[sl] = pd_v[sl] + plsc.load_gather(base_v, [e_v[sl]])
    pltpu.sync_copy(d_v, dest_hbm.at[pl.ds(p0, CHUNK)])
    descs = {}
    for j in range(NSUB):
        b = j % 2
        if j >= 2:
            for cp in descs[j - 2]:
                cp.wait()
        t0 = pl.multiple_of(wid * (CHUNK // 2) + j * (SUB // 2), SUB // 2)
        pltpu.sync_copy(x_hbm.at[pl.ds(t0, SUB // 2)], rows_v[b])
        for c in range(SUB // 32):
            sl = pl.ds(c * 16, 16)
            ii = lax.iota(jnp.int32, 16) * 2 + j * SUB + c * 32
            de_v[b][sl] = plsc.load_gather(d_v, [ii])
            do_v[b][sl] = plsc.load_gather(d_v, [ii + 1])
        cp1 = pltpu.async_copy(rows_v[b], xs_hbm.at[de_v[b]], seme[b])
        cp2 = pltpu.async_copy(rows_v[b], xs_hbm.at[do_v[b]], semo[b])
        descs[j] = (cp1, cp2)
    for j in (NSUB - 2, NSUB - 1):
        for cp in descs[j]:
            cp.wait()


_dispatch = functools.partial(
    pl.kernel,
    _dispatch_body,
    compiler_params=pltpu.CompilerParams(needs_layout_passes=False),
    out_type=[
        jax.ShapeDtypeStruct((CAP, Dm // 2), jnp.float32),
        jax.ShapeDtypeStruct((PAIRS,), jnp.int32),
    ],
    scratch_types=[
        pltpu.VMEM((16,), jnp.int32),
        pltpu.VMEM((CHUNK,), jnp.int32),
        pltpu.VMEM((CHUNK,), jnp.int32),
        pltpu.VMEM((CHUNK,), jnp.int32),
        [pltpu.VMEM((SUB // 2,), jnp.int32)] * 2,
        [pltpu.VMEM((SUB // 2,), jnp.int32)] * 2,
        [pltpu.VMEM((SUB // 2, Dm // 2), jnp.float32)] * 2,
        [pltpu.SemaphoreType.DMA] * 2,
        [pltpu.SemaphoreType.DMA] * 2,
    ],
)


def _emap(j, cum):
    e = jnp.int32(0)
    for k in range(E):
        e = e + (j >= cum[k]).astype(jnp.int32)
    return jnp.minimum(e, E - 1)


def _gemm_body(cum_ref, xs_ref, w1a_ref, w1b_ref, b1_ref, w2_ref, b2_ref,
               ys_ref):
    j = pl.program_id(0)

    @pl.when(j < cum_ref[E - 1])
    def _():
        pu = lax.bitcast_convert_type(xs_ref[...], jnp.uint32)
        xa = lax.bitcast_convert_type((pu & 0xFFFF).astype(jnp.uint16),
                                      jnp.bfloat16)
        xb2 = lax.bitcast_convert_type((pu >> 16).astype(jnp.uint16),
                                       jnp.bfloat16)
        h = (lax.dot_general(xa, w1a_ref[0], (((1,), (1,)), ((), ())),
                             preferred_element_type=jnp.float32)
             + lax.dot_general(xb2, w1b_ref[0], (((1,), (1,)), ((), ())),
                               preferred_element_type=jnp.float32)
             + b1_ref[0])
        h = jnp.maximum(h, 0.0).astype(jnp.bfloat16)
        ys_ref[...] = lax.dot_general(h, w2_ref[0], (((1,), (1,)), ((), ())),
                                      preferred_element_type=jnp.float32) + b2_ref[0]


def _gemm(cumt, xs, W1a, W1bh, b1r, W2b, b2r):
    grid_spec = pltpu.PrefetchScalarGridSpec(
        num_scalar_prefetch=1,
        grid=(MAX_TILES,),
        in_specs=[
            pl.BlockSpec((TBg, Dm // 2), lambda j, cum: (j, 0)),
            pl.BlockSpec((1, H, Dm // 2), lambda j, cum: (_emap(j, cum), 0, 0)),
            pl.BlockSpec((1, H, Dm // 2), lambda j, cum: (_emap(j, cum), 0, 0)),
            pl.BlockSpec((1, 1, H), lambda j, cum: (_emap(j, cum), 0, 0)),
            pl.BlockSpec((1, O, H), lambda j, cum: (_emap(j, cum), 0, 0)),
            pl.BlockSpec((1, 1, O), lambda j, cum: (_emap(j, cum), 0, 0)),
        ],
        out_specs=pl.BlockSpec((TBg, O), lambda j, cum: (j, 0)),
    )
    return pl.pallas_call(
        _gemm_body,
        grid_spec=grid_spec,
        out_shape=jax.ShapeDtypeStruct((CAP, O), jnp.float32),
    )(cumt, xs, W1a, W1bh, b1r, W2b, b2r)


NSUBC = TCHUNK // STOK


def _combine_body(ys_hbm, dest_hbm, gates_hbm, out_hbm, d_all, g_all, de_v,
                  do_v, o_v, r_v, sg, sr, so):
    wid = lax.axis_index("s") * 2 + lax.axis_index("c")
    q0 = pl.multiple_of(wid * 2 * TCHUNK, 2 * TCHUNK)
    pltpu.sync_copy(dest_hbm.at[pl.ds(q0, 2 * TCHUNK)], d_all)

    def stage(j, b):
        for c in range(STOK // 16):
            sl = pl.ds(c * 16, 16)
            ii = lax.iota(jnp.int32, 16) * 2 + j * 2 * STOK + c * 32
            de_v[b][sl] = plsc.load_gather(d_all, [ii])
            do_v[b][sl] = plsc.load_gather(d_all, [ii + 1])
        pltpu.sync_copy(
            gates_hbm.at[pl.ds(q0 + j * 2 * STOK, 2 * STOK)], g_all[b])
        ge = pltpu.async_copy(ys_hbm.at[de_v[b]], o_v[b], sg[b])
        go = pltpu.async_copy(ys_hbm.at[do_v[b]], r_v[b], sr[b])
        return ge, go

    gd = {}
    od = {}
    gd[0] = stage(0, 0)
    for j in range(NSUBC):
        b = j % 2
        if j + 1 < NSUBC:
            if j >= 1:
                od[j - 1].wait()
            gd[j + 1] = stage(j + 1, 1 - b)
        gd[j][0].wait()
        gd[j][1].wait()

        def body(t, carry):
            g0 = g_all[b][2 * t, :]
            g1 = g_all[b][2 * t + 1, :]
            for cc in range(Dm // 16):
                sl = pl.ds(cc * 16, 16)
                o_v[b][t, sl] = g0 * o_v[b][t, sl] + g1 * r_v[b][t, sl]
            return carry

        lax.fori_loop(0, STOK, body, 0)
        t0 = pl.multiple_of(wid * TCHUNK + j * STOK, STOK)
        od[j] = pltpu.async_copy(o_v[b], out_hbm.at[pl.ds(t0, STOK)], so[b])
    od[NSUBC - 2].wait()
    od[NSUBC - 1].wait()


_combine = functools.partial(
    pl.kernel,
    _combine_body,
    compiler_params=pltpu.CompilerParams(needs_layout_passes=False),
    out_type=jax.ShapeDtypeStruct((T, Dm), jnp.float32),
    scratch_types=[
        pltpu.VMEM((2 * TCHUNK,), jnp.int32),
        [pltpu.VMEM((2 * STOK, 16), jnp.float32)] * 2,
        [pltpu.VMEM((STOK,), jnp.int32)] * 2,
        [pltpu.VMEM((STOK,), jnp.int32)] * 2,
        [pltpu.VMEM((STOK, Dm), jnp.float32)] * 2,
        [pltpu.VMEM((STOK, Dm), jnp.float32)] * 2,
        [pltpu.SemaphoreType.DMA] * 2,
        [pltpu.SemaphoreType.DMA] * 2,
        [pltpu.SemaphoreType.DMA] * 2,
    ],
)


def kernel(x, W1, b1, W2, b2, Wr, br):
    xf = x.reshape(T, Dm)
    ints, flts, base_o, cumt_o, xp = _router(xf, Wr, br.reshape(1, E))
    eids = ints[:, 0:2].reshape(PAIRS)
    pdest = ints[:, 2:4].reshape(PAIRS)
    gates16 = jnp.broadcast_to(flts.reshape(PAIRS, 1), (PAIRS, 16))
    base16 = jnp.concatenate(
        [base_o.reshape(E), jnp.zeros((16 - E,), jnp.int32)])
    cumt = cumt_o.reshape(E)

    mesh = plsc.VectorSubcoreMesh(core_axis_name="c", subcore_axis_name="s")
    xs, dest = _dispatch(mesh=mesh)(xp, pdest, eids, base16)
    W1bf = W1.astype(jnp.bfloat16)
    ys = _gemm(cumt, xs, W1bf[:, :, :Dm // 2], W1bf[:, :, Dm // 2:],
               b1.reshape(E, 1, H), W2.astype(jnp.bfloat16),
               b2.reshape(E, 1, O))
    out = _combine(mesh=mesh)(ys, dest, gates16)
    return out.reshape(B, N_OBJ, O)


# speedup vs baseline: 1.1035x; 1.0304x over previous
import functools

import jax
import jax.numpy as jnp
from jax import lax
from jax.experimental import pallas as pl
from jax.experimental.pallas import tpu as pltpu
from jax.experimental.pallas import tpu_sc as plsc

B, N_OBJ, Dm = 4, 2048, 768
E = 8
H = 768
O = 768
T = B * N_OBJ
PAIRS = 2 * T
TB = 512
NB = T // TB
TBg = 256
CAP = PAIRS + E * TBg
MAX_TILES = CAP // TBg

NW = 32
CHUNK = PAIRS // NW
SUB = 64
TCHUNK = T // NW
STOK = 16


def _router_body(x_ref, Wr_ref, br_ref, ints_ref, flts_ref, base_ref, cumt_ref,
                 xp_ref, run_ref):
    pid = pl.program_id(0)

    @pl.when(pid == 0)
    def _():
        run_ref[...] = jnp.zeros((1, E), jnp.float32)

    xb = x_ref[...]
    au = lax.bitcast_convert_type(
        xb[:, :Dm // 2].astype(jnp.bfloat16), jnp.uint16).astype(jnp.uint32)
    bu = lax.bitcast_convert_type(
        xb[:, Dm // 2:].astype(jnp.bfloat16), jnp.uint16).astype(jnp.uint32)
    xp_ref[...] = lax.bitcast_convert_type(au | (bu << 16), jnp.float32)
    logits = lax.dot_general(xb, Wr_ref[...], (((1,), (1,)), ((), ())),
                             preferred_element_type=jnp.float32) + br_ref[...]
    ids = lax.broadcasted_iota(jnp.int32, (TB, E), 1)
    m0 = jnp.max(logits, axis=1, keepdims=True)
    a0 = jnp.min(jnp.where(logits == m0, ids, E), axis=1, keepdims=True)
    l1 = jnp.where(ids == a0, -jnp.inf, logits)
    m1 = jnp.max(l1, axis=1, keepdims=True)
    a1 = jnp.min(jnp.where(l1 == m1, ids, E), axis=1, keepdims=True)
    c0 = 1.0 / (1.0 + jnp.exp(m1 - m0))
    c1 = 1.0 - c0

    oh0 = (ids == a0).astype(jnp.float32)
    oh1 = (ids == a1).astype(jnp.float32)
    Hh = oh0 + oh1
    ri = lax.broadcasted_iota(jnp.int32, (TB, TB), 0)
    ci = lax.broadcasted_iota(jnp.int32, (TB, TB), 1)
    Ltri = (ci < ri).astype(jnp.float32)
    run = run_ref[...]
    C = lax.dot_general(Ltri, Hh, (((1,), (0,)), ((), ())),
                        preferred_element_type=jnp.float32) + run
    r0 = jnp.sum(C * oh0, axis=1, keepdims=True)
    r1 = jnp.sum(C * oh1, axis=1, keepdims=True)

    ints_ref[...] = jnp.concatenate(
        [a0, a1, r0.astype(jnp.int32), r1.astype(jnp.int32)], axis=1)
    flts_ref[...] = jnp.concatenate([c0, c1], axis=1)

    newrun = run + jnp.sum(Hh, axis=0, keepdims=True)
    run_ref[...] = newrun

    @pl.when(pid == NB - 1)
    def _():
        n = jnp.floor((newrun + (TBg - 1)) / TBg)
        p = n * TBg
        el = lax.broadcasted_iota(jnp.int32, (E, E), 0)
        ec = lax.broadcasted_iota(jnp.int32, (E, E), 1)
        Mstrict = (el < ec).astype(jnp.float32)
        Mle = (el <= ec).astype(jnp.float32)
        base_ref[...] = lax.dot_general(
            p, Mstrict, (((1,), (0,)), ((), ())),
            preferred_element_type=jnp.float32).astype(jnp.int32)
        cumt_ref[...] = lax.dot_general(
            n, Mle, (((1,), (0,)), ((), ())),
            preferred_element_type=jnp.float32).astype(jnp.int32)


def _router(xf, Wr, brr):
    return pl.pallas_call(
        _router_body,
        grid=(NB,),
        in_specs=[
            pl.BlockSpec((TB, Dm), lambda i: (i, 0)),
            pl.BlockSpec((E, Dm), lambda i: (0, 0)),
            pl.BlockSpec((1, E), lambda i: (0, 0)),
        ],
        out_specs=[
            pl.BlockSpec((TB, 4), lambda i: (i, 0)),
            pl.BlockSpec((TB, 2), lambda i: (i, 0)),
            pl.BlockSpec((1, E), lambda i: (0, 0)),
            pl.BlockSpec((1, E), lambda i: (0, 0)),
            pl.BlockSpec((TB, Dm // 2), lambda i: (i, 0)),
        ],
        out_shape=[
            jax.ShapeDtypeStruct((T, 4), jnp.int32),
            jax.ShapeDtypeStruct((T, 2), jnp.float32),
            jax.ShapeDtypeStruct((1, E), jnp.int32),
            jax.ShapeDtypeStruct((1, E), jnp.int32),
            jax.ShapeDtypeStruct((T, Dm // 2), jnp.float32),
        ],
        scratch_shapes=[pltpu.VMEM((1, E), jnp.float32)],
    )(xf, Wr, brr)


NSUB = CHUNK // SUB


def _dispatch_body(x_hbm, pdest_hbm, eids_hbm, base_hbm, xs_hbm,
                   dest_hbm, base_v, pd_v, e_v, d_v, de_v, do_v,
                   rows_v, seme, semo):
    wid = lax.axis_index("s") * 2 + lax.axis_index("c")
    p0 = pl.multiple_of(wid * CHUNK, CHUNK)
    pltpu.sync_copy(base_hbm, base_v)
    pltpu.sync_copy(pdest_hbm.at[pl.ds(p0, CHUNK)], pd_v)
    pltpu.sync_copy(eids_hbm.at[pl.ds(p0, CHUNK)], e_v)
    for c in range(CHUNK // 16):
        sl = pl.ds(c * 16, 16)
        d_v[sl] = pd_v[sl] + plsc.load_gather(base_v, [e_v[sl]])
    pltpu.sync_copy(d_v, dest_hbm.at[pl.ds(p0, CHUNK)])
    descs = {}
    for j in range(NSUB):
        b = j % 2
        if j >= 2:
            for cp in descs[j - 2]:
                cp.wait()
        t0 = pl.multiple_of(wid * (CHUNK // 2) + j * (SUB // 2), SUB // 2)
        pltpu.sync_copy(x_hbm.at[pl.ds(t0, SUB // 2)], rows_v[b])
        for c in range(SUB // 32):
            sl = pl.ds(c * 16, 16)
            ii = lax.iota(jnp.int32, 16) * 2 + j * SUB + c * 32
            de_v[b][sl] = plsc.load_gather(d_v, [ii])
            do_v[b][sl] = plsc.load_gather(d_v, [ii + 1])
        cp1 = pltpu.async_copy(rows_v[b], xs_hbm.at[de_v[b]], seme[b])
        cp2 = pltpu.async_copy(rows_v[b], xs_hbm.at[do_v[b]], semo[b])
        descs[j] = (cp1, cp2)
    for j in (NSUB - 2, NSUB - 1):
        for cp in descs[j]:
            cp.wait()


_dispatch = functools.partial(
    pl.kernel,
    _dispatch_body,
    compiler_params=pltpu.CompilerParams(needs_layout_passes=False),
    out_type=[
        jax.ShapeDtypeStruct((CAP, Dm // 2), jnp.float32),
        jax.ShapeDtypeStruct((PAIRS,), jnp.int32),
    ],
    scratch_types=[
        pltpu.VMEM((16,), jnp.int32),
        pltpu.VMEM((CHUNK,), jnp.int32),
        pltpu.VMEM((CHUNK,), jnp.int32),
        pltpu.VMEM((CHUNK,), jnp.int32),
        [pltpu.VMEM((SUB // 2,), jnp.int32)] * 2,
        [pltpu.VMEM((SUB // 2,), jnp.int32)] * 2,
        [pltpu.VMEM((SUB // 2, Dm // 2), jnp.float32)] * 2,
        [pltpu.SemaphoreType.DMA] * 2,
        [pltpu.SemaphoreType.DMA] * 2,
    ],
)


def _emap(j, cum):
    e = jnp.int32(0)
    for k in range(E):
        e = e + (j >= cum[k]).astype(jnp.int32)
    return jnp.minimum(e, E - 1)


def _gemm_body(cum_ref, xs_ref, w1a_ref, w1b_ref, b1_ref, w2_ref, b2_ref,
               ys_ref):
    j = pl.program_id(0)

    @pl.when(j < cum_ref[E - 1])
    def _():
        pu = lax.bitcast_convert_type(xs_ref[...], jnp.uint32)
        xa = lax.bitcast_convert_type((pu & 0xFFFF).astype(jnp.uint16),
                                      jnp.bfloat16)
        xb2 = lax.bitcast_convert_type((pu >> 16).astype(jnp.uint16),
                                       jnp.bfloat16)
        h = (lax.dot_general(xa, w1a_ref[0], (((1,), (1,)), ((), ())),
                             preferred_element_type=jnp.float32)
             + lax.dot_general(xb2, w1b_ref[0], (((1,), (1,)), ((), ())),
                               preferred_element_type=jnp.float32)
             + b1_ref[0])
        h = jnp.maximum(h, 0.0).astype(jnp.bfloat16)
        ys_ref[...] = lax.dot_general(h, w2_ref[0], (((1,), (1,)), ((), ())),
                                      preferred_element_type=jnp.float32) + b2_ref[0]


def _gemm(cumt, xs, W1a, W1bh, b1r, W2b, b2r):
    grid_spec = pltpu.PrefetchScalarGridSpec(
        num_scalar_prefetch=1,
        grid=(MAX_TILES,),
        in_specs=[
            pl.BlockSpec((TBg, Dm // 2), lambda j, cum: (j, 0)),
            pl.BlockSpec((1, H, Dm // 2), lambda j, cum: (_emap(j, cum), 0, 0)),
            pl.BlockSpec((1, H, Dm // 2), lambda j, cum: (_emap(j, cum), 0, 0)),
            pl.BlockSpec((1, 1, H), lambda j, cum: (_emap(j, cum), 0, 0)),
            pl.BlockSpec((1, O, H), lambda j, cum: (_emap(j, cum), 0, 0)),
            pl.BlockSpec((1, 1, O), lambda j, cum: (_emap(j, cum), 0, 0)),
        ],
        out_specs=pl.BlockSpec((TBg, O), lambda j, cum: (j, 0)),
    )
    return pl.pallas_call(
        _gemm_body,
        grid_spec=grid_spec,
        out_shape=jax.ShapeDtypeStruct((CAP, O), jnp.float32),
    )(cumt, xs, W1a, W1bh, b1r, W2b, b2r)


NSUBC = TCHUNK // STOK


def _combine_body(ys_hbm, dest_hbm, gates_hbm, out_hbm, d_all, g_all, de_v,
                  do_v, o_v, r_v, sg, sr, so):
    wid = lax.axis_index("s") * 2 + lax.axis_index("c")
    q0 = pl.multiple_of(wid * 2 * TCHUNK, 2 * TCHUNK)
    pltpu.sync_copy(dest_hbm.at[pl.ds(q0, 2 * TCHUNK)], d_all)
    pltpu.sync_copy(gates_hbm.at[pl.ds(q0, 2 * TCHUNK)], g_all)

    def stage(j, b):
        for c in range(STOK // 16):
            sl = pl.ds(c * 16, 16)
            ii = lax.iota(jnp.int32, 16) * 2 + j * 2 * STOK + c * 32
            de_v[b][sl] = plsc.load_gather(d_all, [ii])
            do_v[b][sl] = plsc.load_gather(d_all, [ii + 1])
        ge = pltpu.async_copy(ys_hbm.at[de_v[b]], o_v[b], sg[b])
        go = pltpu.async_copy(ys_hbm.at[do_v[b]], r_v[b], sr[b])
        return ge, go

    gd = {}
    od = {}
    gd[0] = stage(0, 0)
    for j in range(NSUBC):
        b = j % 2
        if j + 1 < NSUBC:
            if j >= 1:
                od[j - 1].wait()
            gd[j + 1] = stage(j + 1, 1 - b)
        gd[j][0].wait()
        gd[j][1].wait()

        def body(t, carry):
            g0 = g_all[j * 2 * STOK + 2 * t, :]
            g1 = g_all[j * 2 * STOK + 2 * t + 1, :]
            for cc in range(Dm // 16):
                sl = pl.ds(cc * 16, 16)
                o_v[b][t, sl] = g0 * o_v[b][t, sl] + g1 * r_v[b][t, sl]
            return carry

        lax.fori_loop(0, STOK, body, 0)
        t0 = pl.multiple_of(wid * TCHUNK + j * STOK, STOK)
        od[j] = pltpu.async_copy(o_v[b], out_hbm.at[pl.ds(t0, STOK)], so[b])
    od[NSUBC - 2].wait()
    od[NSUBC - 1].wait()


_combine = functools.partial(
    pl.kernel,
    _combine_body,
    compiler_params=pltpu.CompilerParams(needs_layout_passes=False),
    out_type=jax.ShapeDtypeStruct((T, Dm), jnp.float32),
    scratch_types=[
        pltpu.VMEM((2 * TCHUNK,), jnp.int32),
        pltpu.VMEM((2 * TCHUNK, 16), jnp.float32),
        [pltpu.VMEM((STOK,), jnp.int32)] * 2,
        [pltpu.VMEM((STOK,), jnp.int32)] * 2,
        [pltpu.VMEM((STOK, Dm), jnp.float32)] * 2,
        [pltpu.VMEM((STOK, Dm), jnp.float32)] * 2,
        [pltpu.SemaphoreType.DMA] * 2,
        [pltpu.SemaphoreType.DMA] * 2,
        [pltpu.SemaphoreType.DMA] * 2,
    ],
)


def kernel(x, W1, b1, W2, b2, Wr, br):
    xf = x.reshape(T, Dm)
    ints, flts, base_o, cumt_o, xp = _router(xf, Wr, br.reshape(1, E))
    eids = ints[:, 0:2].reshape(PAIRS)
    pdest = ints[:, 2:4].reshape(PAIRS)
    gates16 = jnp.broadcast_to(flts.reshape(PAIRS, 1), (PAIRS, 16))
    base16 = jnp.concatenate(
        [base_o.reshape(E), jnp.zeros((16 - E,), jnp.int32)])
    cumt = cumt_o.reshape(E)

    mesh = plsc.VectorSubcoreMesh(core_axis_name="c", subcore_axis_name="s")
    xs, dest = _dispatch(mesh=mesh)(xp, pdest, eids, base16)
    W1bf = W1.astype(jnp.bfloat16)
    ys = _gemm(cumt, xs, W1bf[:, :, :Dm // 2], W1bf[:, :, Dm // 2:],
               b1.reshape(E, 1, H), W2.astype(jnp.bfloat16),
               b2.reshape(E, 1, O))
    out = _combine(mesh=mesh)(ys, dest, gates16)
    return out.reshape(B, N_OBJ, O)


# speedup vs baseline: 1.1788x; 1.0682x over previous
import functools

import jax
import jax.numpy as jnp
from jax import lax
from jax.experimental import pallas as pl
from jax.experimental.pallas import tpu as pltpu
from jax.experimental.pallas import tpu_sc as plsc

B, N_OBJ, Dm = 4, 2048, 768
E = 8
H = 768
O = 768
T = B * N_OBJ
PAIRS = 2 * T
TB = 512
NB = T // TB
TBg = 512
CAP = PAIRS + E * TBg
MAX_TILES = CAP // TBg

NW = 32
CHUNK = PAIRS // NW
SUB = 64
TCHUNK = T // NW
STOK = 16


def _router_body(x_ref, Wr_ref, br_ref, ints_ref, flts_ref, base_ref, cumt_ref,
                 xp_ref, run_ref):
    pid = pl.program_id(0)

    @pl.when(pid == 0)
    def _():
        run_ref[...] = jnp.zeros((1, E), jnp.float32)

    xb = x_ref[...]
    au = lax.bitcast_convert_type(
        xb[:, :Dm // 2].astype(jnp.bfloat16), jnp.uint16).astype(jnp.uint32)
    bu = lax.bitcast_convert_type(
        xb[:, Dm // 2:].astype(jnp.bfloat16), jnp.uint16).astype(jnp.uint32)
    xp_ref[...] = lax.bitcast_convert_type(au | (bu << 16), jnp.float32)
    logits = lax.dot_general(xb, Wr_ref[...], (((1,), (1,)), ((), ())),
                             preferred_element_type=jnp.float32) + br_ref[...]
    ids = lax.broadcasted_iota(jnp.int32, (TB, E), 1)
    m0 = jnp.max(logits, axis=1, keepdims=True)
    a0 = jnp.min(jnp.where(logits == m0, ids, E), axis=1, keepdims=True)
    l1 = jnp.where(ids == a0, -jnp.inf, logits)
    m1 = jnp.max(l1, axis=1, keepdims=True)
    a1 = jnp.min(jnp.where(l1 == m1, ids, E), axis=1, keepdims=True)
    c0 = 1.0 / (1.0 + jnp.exp(m1 - m0))
    c1 = 1.0 - c0

    oh0 = (ids == a0).astype(jnp.float32)
    oh1 = (ids == a1).astype(jnp.float32)
    Hh = oh0 + oh1
    ri = lax.broadcasted_iota(jnp.int32, (TB, TB), 0)
    ci = lax.broadcasted_iota(jnp.int32, (TB, TB), 1)
    Ltri = (ci < ri).astype(jnp.float32)
    run = run_ref[...]
    C = lax.dot_general(Ltri, Hh, (((1,), (0,)), ((), ())),
                        preferred_element_type=jnp.float32) + run
    r0 = jnp.sum(C * oh0, axis=1, keepdims=True)
    r1 = jnp.sum(C * oh1, axis=1, keepdims=True)

    ints_ref[...] = jnp.concatenate(
        [a0, a1, r0.astype(jnp.int32), r1.astype(jnp.int32)], axis=1)
    flts_ref[...] = jnp.concatenate([c0, c1], axis=1)

    newrun = run + jnp.sum(Hh, axis=0, keepdims=True)
    run_ref[...] = newrun

    @pl.when(pid == NB - 1)
    def _():
        n = jnp.floor((newrun + (TBg - 1)) / TBg)
        p = n * TBg
        el = lax.broadcasted_iota(jnp.int32, (E, E), 0)
        ec = lax.broadcasted_iota(jnp.int32, (E, E), 1)
        Mstrict = (el < ec).astype(jnp.float32)
        Mle = (el <= ec).astype(jnp.float32)
        base_ref[...] = lax.dot_general(
            p, Mstrict, (((1,), (0,)), ((), ())),
            preferred_element_type=jnp.float32).astype(jnp.int32)
        cumt_ref[...] = lax.dot_general(
            n, Mle, (((1,), (0,)), ((), ())),
            preferred_element_type=jnp.float32).astype(jnp.int32)


def _router(xf, Wr, brr):
    return pl.pallas_call(
        _router_body,
        grid=(NB,),
        in_specs=[
            pl.BlockSpec((TB, Dm), lambda i: (i, 0)),
            pl.BlockSpec((E, Dm), lambda i: (0, 0)),
            pl.BlockSpec((1, E), lambda i: (0, 0)),
        ],
        out_specs=[
            pl.BlockSpec((TB, 4), lambda i: (i, 0)),
            pl.BlockSpec((TB, 2), lambda i: (i, 0)),
            pl.BlockSpec((1, E), lambda i: (0, 0)),
            pl.BlockSpec((1, E), lambda i: (0, 0)),
            pl.BlockSpec((TB, Dm // 2), lambda i: (i, 0)),
        ],
        out_shape=[
            jax.ShapeDtypeStruct((T, 4), jnp.int32),
            jax.ShapeDtypeStruct((T, 2), jnp.float32),
            jax.ShapeDtypeStruct((1, E), jnp.int32),
            jax.ShapeDtypeStruct((1, E), jnp.int32),
            jax.ShapeDtypeStruct((T, Dm // 2), jnp.float32),
        ],
        scratch_shapes=[pltpu.VMEM((1, E), jnp.float32)],
    )(xf, Wr, brr)


NSUB = CHUNK // SUB


def _dispatch_body(x_hbm, pdest_hbm, eids_hbm, base_hbm, xs_hbm,
                   dest_hbm, base_v, pd_v, e_v, d_v, de_v, do_v,
                   rows_v, seme, semo):
    wid = lax.axis_index("s") * 2 + lax.axis_index("c")
    p0 = pl.multiple_of(wid * CHUNK, CHUNK)
    pltpu.sync_copy(base_hbm, base_v)
    pltpu.sync_copy(pdest_hbm.at[pl.ds(p0, CHUNK)], pd_v)
    pltpu.sync_copy(eids_hbm.at[pl.ds(p0, CHUNK)], e_v)
    for c in range(CHUNK // 16):
        sl = pl.ds(c * 16, 16)
        d_v[sl] = pd_v[sl] + plsc.load_gather(base_v, [e_v[sl]])
    pltpu.sync_copy(d_v, dest_hbm.at[pl.ds(p0, CHUNK)])
    descs = {}
    for j in range(NSUB):
        b = j % 2
        if j >= 2:
            for cp in descs[j - 2]:
                cp.wait()
        t0 = pl.multiple_of(wid * (CHUNK // 2) + j * (SUB // 2), SUB // 2)
        pltpu.sync_copy(x_hbm.at[pl.ds(t0, SUB // 2)], rows_v[b])
        for c in range(SUB // 32):
            sl = pl.ds(c * 16, 16)
            ii = lax.iota(jnp.int32, 16) * 2 + j * SUB + c * 32
            de_v[b][sl] = plsc.load_gather(d_v, [ii])
            do_v[b][sl] = plsc.load_gather(d_v, [ii + 1])
        cp1 = pltpu.async_copy(rows_v[b], xs_hbm.at[de_v[b]], seme[b])
        cp2 = pltpu.async_copy(rows_v[b], xs_hbm.at[do_v[b]], semo[b])
        descs[j] = (cp1, cp2)
    for j in (NSUB - 2, NSUB - 1):
        for cp in descs[j]:
            cp.wait()


_dispatch = functools.partial(
    pl.kernel,
    _dispatch_body,
    compiler_params=pltpu.CompilerParams(needs_layout_passes=False),
    out_type=[
        jax.ShapeDtypeStruct((CAP, Dm // 2), jnp.float32),
        jax.ShapeDtypeStruct((PAIRS,), jnp.int32),
    ],
    scratch_types=[
        pltpu.VMEM((16,), jnp.int32),
        pltpu.VMEM((CHUNK,), jnp.int32),
        pltpu.VMEM((CHUNK,), jnp.int32),
        pltpu.VMEM((CHUNK,), jnp.int32),
        [pltpu.VMEM((SUB // 2,), jnp.int32)] * 2,
        [pltpu.VMEM((SUB // 2,), jnp.int32)] * 2,
        [pltpu.VMEM((SUB // 2, Dm // 2), jnp.float32)] * 2,
        [pltpu.SemaphoreType.DMA] * 2,
        [pltpu.SemaphoreType.DMA] * 2,
    ],
)


def _emap(j, cum):
    e = jnp.int32(0)
    for k in range(E):
        e = e + (j >= cum[k]).astype(jnp.int32)
    return jnp.minimum(e, E - 1)


def _gemm_body(cum_ref, xs_ref, w1a_ref, w1b_ref, b1_ref, w2_ref, b2_ref,
               ys_ref):
    j = pl.program_id(0)

    @pl.when(j < cum_ref[E - 1])
    def _():
        pu = lax.bitcast_convert_type(xs_ref[...], jnp.uint32)
        xa = lax.bitcast_convert_type((pu & 0xFFFF).astype(jnp.uint16),
                                      jnp.bfloat16)
        xb2 = lax.bitcast_convert_type((pu >> 16).astype(jnp.uint16),
                                       jnp.bfloat16)
        h = (lax.dot_general(xa, w1a_ref[0], (((1,), (1,)), ((), ())),
                             preferred_element_type=jnp.float32)
             + lax.dot_general(xb2, w1b_ref[0], (((1,), (1,)), ((), ())),
                               preferred_element_type=jnp.float32)
             + b1_ref[0])
        h = jnp.maximum(h, 0.0).astype(jnp.bfloat16)
        ys_ref[...] = lax.dot_general(h, w2_ref[0], (((1,), (1,)), ((), ())),
                                      preferred_element_type=jnp.float32) + b2_ref[0]


def _gemm(cumt, xs, W1a, W1bh, b1r, W2b, b2r):
    grid_spec = pltpu.PrefetchScalarGridSpec(
        num_scalar_prefetch=1,
        grid=(MAX_TILES,),
        in_specs=[
            pl.BlockSpec((TBg, Dm // 2), lambda j, cum: (j, 0)),
            pl.BlockSpec((1, H, Dm // 2), lambda j, cum: (_emap(j, cum), 0, 0)),
            pl.BlockSpec((1, H, Dm // 2), lambda j, cum: (_emap(j, cum), 0, 0)),
            pl.BlockSpec((1, 1, H), lambda j, cum: (_emap(j, cum), 0, 0)),
            pl.BlockSpec((1, O, H), lambda j, cum: (_emap(j, cum), 0, 0)),
            pl.BlockSpec((1, 1, O), lambda j, cum: (_emap(j, cum), 0, 0)),
        ],
        out_specs=pl.BlockSpec((TBg, O), lambda j, cum: (j, 0)),
    )
    return pl.pallas_call(
        _gemm_body,
        grid_spec=grid_spec,
        out_shape=jax.ShapeDtypeStruct((CAP, O), jnp.float32),
    )(cumt, xs, W1a, W1bh, b1r, W2b, b2r)


NSUBC = TCHUNK // STOK


def _combine_body(ys_hbm, dest_hbm, gates_hbm, out_hbm, d_all, g_all, de_v,
                  do_v, o_v, r_v, sg, sr, so):
    wid = lax.axis_index("s") * 2 + lax.axis_index("c")
    q0 = pl.multiple_of(wid * 2 * TCHUNK, 2 * TCHUNK)
    pltpu.sync_copy(dest_hbm.at[pl.ds(q0, 2 * TCHUNK)], d_all)
    pltpu.sync_copy(gates_hbm.at[pl.ds(q0, 2 * TCHUNK)], g_all)

    def stage(j, b):
        for c in range(STOK // 16):
            sl = pl.ds(c * 16, 16)
            ii = lax.iota(jnp.int32, 16) * 2 + j * 2 * STOK + c * 32
            de_v[b][sl] = plsc.load_gather(d_all, [ii])
            do_v[b][sl] = plsc.load_gather(d_all, [ii + 1])
        ge = pltpu.async_copy(ys_hbm.at[de_v[b]], o_v[b], sg[b])
        go = pltpu.async_copy(ys_hbm.at[do_v[b]], r_v[b], sr[b])
        return ge, go

    gd = {}
    od = {}
    gd[0] = stage(0, 0)
    for j in range(NSUBC):
        b = j % 2
        if j + 1 < NSUBC:
            if j >= 1:
                od[j - 1].wait()
            gd[j + 1] = stage(j + 1, 1 - b)
        gd[j][0].wait()
        gd[j][1].wait()

        def body(t, carry):
            g0 = g_all[j * 2 * STOK + 2 * t, :]
            g1 = g_all[j * 2 * STOK + 2 * t + 1, :]
            for cc in range(Dm // 16):
                sl = pl.ds(cc * 16, 16)
                o_v[b][t, sl] = g0 * o_v[b][t, sl] + g1 * r_v[b][t, sl]
            return carry

        lax.fori_loop(0, STOK, body, 0)
        t0 = pl.multiple_of(wid * TCHUNK + j * STOK, STOK)
        od[j] = pltpu.async_copy(o_v[b], out_hbm.at[pl.ds(t0, STOK)], so[b])
    od[NSUBC - 2].wait()
    od[NSUBC - 1].wait()


_combine = functools.partial(
    pl.kernel,
    _combine_body,
    compiler_params=pltpu.CompilerParams(needs_layout_passes=False),
    out_type=jax.ShapeDtypeStruct((T, Dm), jnp.float32),
    scratch_types=[
        pltpu.VMEM((2 * TCHUNK,), jnp.int32),
        pltpu.VMEM((2 * TCHUNK, 16), jnp.float32),
        [pltpu.VMEM((STOK,), jnp.int32)] * 2,
        [pltpu.VMEM((STOK,), jnp.int32)] * 2,
        [pltpu.VMEM((STOK, Dm), jnp.float32)] * 2,
        [pltpu.VMEM((STOK, Dm), jnp.float32)] * 2,
        [pltpu.SemaphoreType.DMA] * 2,
        [pltpu.SemaphoreType.DMA] * 2,
        [pltpu.SemaphoreType.DMA] * 2,
    ],
)


def kernel(x, W1, b1, W2, b2, Wr, br):
    xf = x.reshape(T, Dm)
    ints, flts, base_o, cumt_o, xp = _router(xf, Wr, br.reshape(1, E))
    eids = ints[:, 0:2].reshape(PAIRS)
    pdest = ints[:, 2:4].reshape(PAIRS)
    gates16 = jnp.broadcast_to(flts.reshape(PAIRS, 1), (PAIRS, 16))
    base16 = jnp.concatenate(
        [base_o.reshape(E), jnp.zeros((16 - E,), jnp.int32)])
    cumt = cumt_o.reshape(E)

    mesh = plsc.VectorSubcoreMesh(core_axis_name="c", subcore_axis_name="s")
    xs, dest = _dispatch(mesh=mesh)(xp, pdest, eids, base16)
    W1bf = W1.astype(jnp.bfloat16)
    ys = _gemm(cumt, xs, W1bf[:, :, :Dm // 2], W1bf[:, :, Dm // 2:],
               b1.reshape(E, 1, H), W2.astype(jnp.bfloat16),
               b2.reshape(E, 1, O))
    out = _combine(mesh=mesh)(ys, dest, gates16)
    return out.reshape(B, N_OBJ, O)


# speedup vs baseline: 1.2045x; 1.0218x over previous
import functools

import jax
import jax.numpy as jnp
from jax import lax
from jax.experimental import pallas as pl
from jax.experimental.pallas import tpu as pltpu
from jax.experimental.pallas import tpu_sc as plsc

B, N_OBJ, Dm = 4, 2048, 768
E = 8
H = 768
O = 768
T = B * N_OBJ
PAIRS = 2 * T
TB = 512
NB = T // TB
TBg = 1024
CAP = PAIRS + E * TBg
MAX_TILES = CAP // TBg

NW = 32
CHUNK = PAIRS // NW
SUB = 64
TCHUNK = T // NW
STOK = 16


def _router_body(x_ref, Wr_ref, br_ref, ints_ref, flts_ref, base_ref, cumt_ref,
                 xp_ref, run_ref):
    pid = pl.program_id(0)

    @pl.when(pid == 0)
    def _():
        run_ref[...] = jnp.zeros((1, E), jnp.float32)

    xb = x_ref[...]
    au = lax.bitcast_convert_type(
        xb[:, :Dm // 2].astype(jnp.bfloat16), jnp.uint16).astype(jnp.uint32)
    bu = lax.bitcast_convert_type(
        xb[:, Dm // 2:].astype(jnp.bfloat16), jnp.uint16).astype(jnp.uint32)
    xp_ref[...] = lax.bitcast_convert_type(au | (bu << 16), jnp.float32)
    logits = lax.dot_general(xb, Wr_ref[...], (((1,), (1,)), ((), ())),
                             preferred_element_type=jnp.float32) + br_ref[...]
    ids = lax.broadcasted_iota(jnp.int32, (TB, E), 1)
    m0 = jnp.max(logits, axis=1, keepdims=True)
    a0 = jnp.min(jnp.where(logits == m0, ids, E), axis=1, keepdims=True)
    l1 = jnp.where(ids == a0, -jnp.inf, logits)
    m1 = jnp.max(l1, axis=1, keepdims=True)
    a1 = jnp.min(jnp.where(l1 == m1, ids, E), axis=1, keepdims=True)
    c0 = 1.0 / (1.0 + jnp.exp(m1 - m0))
    c1 = 1.0 - c0

    oh0 = (ids == a0).astype(jnp.float32)
    oh1 = (ids == a1).astype(jnp.float32)
    Hh = oh0 + oh1
    ri = lax.broadcasted_iota(jnp.int32, (TB, TB), 0)
    ci = lax.broadcasted_iota(jnp.int32, (TB, TB), 1)
    Ltri = (ci < ri).astype(jnp.float32)
    run = run_ref[...]
    C = lax.dot_general(Ltri, Hh, (((1,), (0,)), ((), ())),
                        preferred_element_type=jnp.float32) + run
    r0 = jnp.sum(C * oh0, axis=1, keepdims=True)
    r1 = jnp.sum(C * oh1, axis=1, keepdims=True)

    ints_ref[...] = jnp.concatenate(
        [a0, a1, r0.astype(jnp.int32), r1.astype(jnp.int32)], axis=1)
    flts_ref[...] = jnp.concatenate([c0, c1], axis=1)

    newrun = run + jnp.sum(Hh, axis=0, keepdims=True)
    run_ref[...] = newrun

    @pl.when(pid == NB - 1)
    def _():
        n = jnp.floor((newrun + (TBg - 1)) / TBg)
        p = n * TBg
        el = lax.broadcasted_iota(jnp.int32, (E, E), 0)
        ec = lax.broadcasted_iota(jnp.int32, (E, E), 1)
        Mstrict = (el < ec).astype(jnp.float32)
        Mle = (el <= ec).astype(jnp.float32)
        base_ref[...] = lax.dot_general(
            p, Mstrict, (((1,), (0,)), ((), ())),
            preferred_element_type=jnp.float32).astype(jnp.int32)
        cumt_ref[...] = lax.dot_general(
            n, Mle, (((1,), (0,)), ((), ())),
            preferred_element_type=jnp.float32).astype(jnp.int32)


def _router(xf, Wr, brr):
    return pl.pallas_call(
        _router_body,
        grid=(NB,),
        in_specs=[
            pl.BlockSpec((TB, Dm), lambda i: (i, 0)),
            pl.BlockSpec((E, Dm), lambda i: (0, 0)),
            pl.BlockSpec((1, E), lambda i: (0, 0)),
        ],
        out_specs=[
            pl.BlockSpec((TB, 4), lambda i: (i, 0)),
            pl.BlockSpec((TB, 2), lambda i: (i, 0)),
            pl.BlockSpec((1, E), lambda i: (0, 0)),
            pl.BlockSpec((1, E), lambda i: (0, 0)),
            pl.BlockSpec((TB, Dm // 2), lambda i: (i, 0)),
        ],
        out_shape=[
            jax.ShapeDtypeStruct((T, 4), jnp.int32),
            jax.ShapeDtypeStruct((T, 2), jnp.float32),
            jax.ShapeDtypeStruct((1, E), jnp.int32),
            jax.ShapeDtypeStruct((1, E), jnp.int32),
            jax.ShapeDtypeStruct((T, Dm // 2), jnp.float32),
        ],
        scratch_shapes=[pltpu.VMEM((1, E), jnp.float32)],
    )(xf, Wr, brr)


NSUB = CHUNK // SUB


def _dispatch_body(x_hbm, pdest_hbm, eids_hbm, base_hbm, xs_hbm,
                   dest_hbm, base_v, pd_v, e_v, d_v, de_v, do_v,
                   rows_v, seme, semo):
    wid = lax.axis_index("s") * 2 + lax.axis_index("c")
    p0 = pl.multiple_of(wid * CHUNK, CHUNK)
    pltpu.sync_copy(base_hbm, base_v)
    pltpu.sync_copy(pdest_hbm.at[pl.ds(p0, CHUNK)], pd_v)
    pltpu.sync_copy(eids_hbm.at[pl.ds(p0, CHUNK)], e_v)
    for c in range(CHUNK // 16):
        sl = pl.ds(c * 16, 16)
        d_v[sl] = pd_v[sl] + plsc.load_gather(base_v, [e_v[sl]])
    pltpu.sync_copy(d_v, dest_hbm.at[pl.ds(p0, CHUNK)])
    descs = {}
    for j in range(NSUB):
        b = j % 2
        if j >= 2:
            for cp in descs[j - 2]:
                cp.wait()
        t0 = pl.multiple_of(wid * (CHUNK // 2) + j * (SUB // 2), SUB // 2)
        pltpu.sync_copy(x_hbm.at[pl.ds(t0, SUB // 2)], rows_v[b])
        for c in range(SUB // 32):
            sl = pl.ds(c * 16, 16)
            ii = lax.iota(jnp.int32, 16) * 2 + j * SUB + c * 32
            de_v[b][sl] = plsc.load_gather(d_v, [ii])
            do_v[b][sl] = plsc.load_gather(d_v, [ii + 1])
        cp1 = pltpu.async_copy(rows_v[b], xs_hbm.at[de_v[b]], seme[b])
        cp2 = pltpu.async_copy(rows_v[b], xs_hbm.at[do_v[b]], semo[b])
        descs[j] = (cp1, cp2)
    for j in (NSUB - 2, NSUB - 1):
        for cp in descs[j]:
            cp.wait()


_dispatch = functools.partial(
    pl.kernel,
    _dispatch_body,
    compiler_params=pltpu.CompilerParams(needs_layout_passes=False),
    out_type=[
        jax.ShapeDtypeStruct((CAP, Dm // 2), jnp.float32),
        jax.ShapeDtypeStruct((PAIRS,), jnp.int32),
    ],
    scratch_types=[
        pltpu.VMEM((16,), jnp.int32),
        pltpu.VMEM((CHUNK,), jnp.int32),
        pltpu.VMEM((CHUNK,), jnp.int32),
        pltpu.VMEM((CHUNK,), jnp.int32),
        [pltpu.VMEM((SUB // 2,), jnp.int32)] * 2,
        [pltpu.VMEM((SUB // 2,), jnp.int32)] * 2,
        [pltpu.VMEM((SUB // 2, Dm // 2), jnp.float32)] * 2,
        [pltpu.SemaphoreType.DMA] * 2,
        [pltpu.SemaphoreType.DMA] * 2,
    ],
)


def _emap(j, cum):
    e = jnp.int32(0)
    for k in range(E):
        e = e + (j >= cum[k]).astype(jnp.int32)
    return jnp.minimum(e, E - 1)


def _gemm_body(cum_ref, xs_ref, w1a_ref, w1b_ref, b1_ref, w2_ref, b2_ref,
               ys_ref):
    j = pl.program_id(0)

    @pl.when(j < cum_ref[E - 1])
    def _():
        pu = lax.bitcast_convert_type(xs_ref[...], jnp.uint32)
        xa = lax.bitcast_convert_type((pu & 0xFFFF).astype(jnp.uint16),
                                      jnp.bfloat16)
        xb2 = lax.bitcast_convert_type((pu >> 16).astype(jnp.uint16),
                                       jnp.bfloat16)
        h = (lax.dot_general(xa, w1a_ref[0], (((1,), (1,)), ((), ())),
                             preferred_element_type=jnp.float32)
             + lax.dot_general(xb2, w1b_ref[0], (((1,), (1,)), ((), ())),
                               preferred_element_type=jnp.float32)
             + b1_ref[0])
        h = jnp.maximum(h, 0.0).astype(jnp.bfloat16)
        ys_ref[...] = lax.dot_general(h, w2_ref[0], (((1,), (1,)), ((), ())),
                                      preferred_element_type=jnp.float32) + b2_ref[0]


def _gemm(cumt, xs, W1a, W1bh, b1r, W2b, b2r):
    grid_spec = pltpu.PrefetchScalarGridSpec(
        num_scalar_prefetch=1,
        grid=(MAX_TILES,),
        in_specs=[
            pl.BlockSpec((TBg, Dm // 2), lambda j, cum: (j, 0)),
            pl.BlockSpec((1, H, Dm // 2), lambda j, cum: (_emap(j, cum), 0, 0)),
            pl.BlockSpec((1, H, Dm // 2), lambda j, cum: (_emap(j, cum), 0, 0)),
            pl.BlockSpec((1, 1, H), lambda j, cum: (_emap(j, cum), 0, 0)),
            pl.BlockSpec((1, O, H), lambda j, cum: (_emap(j, cum), 0, 0)),
            pl.BlockSpec((1, 1, O), lambda j, cum: (_emap(j, cum), 0, 0)),
        ],
        out_specs=pl.BlockSpec((TBg, O), lambda j, cum: (j, 0)),
    )
    return pl.pallas_call(
        _gemm_body,
        grid_spec=grid_spec,
        out_shape=jax.ShapeDtypeStruct((CAP, O), jnp.float32),
    )(cumt, xs, W1a, W1bh, b1r, W2b, b2r)


NSUBC = TCHUNK // STOK


def _combine_body(ys_hbm, dest_hbm, gates_hbm, out_hbm, d_all, g_all, de_v,
                  do_v, o_v, r_v, sg, sr, so):
    wid = lax.axis_index("s") * 2 + lax.axis_index("c")
    q0 = pl.multiple_of(wid * 2 * TCHUNK, 2 * TCHUNK)
    pltpu.sync_copy(dest_hbm.at[pl.ds(q0, 2 * TCHUNK)], d_all)
    pltpu.sync_copy(gates_hbm.at[pl.ds(q0, 2 * TCHUNK)], g_all)

    def stage(j, b):
        for c in range(STOK // 16):
            sl = pl.ds(c * 16, 16)
            ii = lax.iota(jnp.int32, 16) * 2 + j * 2 * STOK + c * 32
            de_v[b][sl] = plsc.load_gather(d_all, [ii])
            do_v[b][sl] = plsc.load_gather(d_all, [ii + 1])
        ge = pltpu.async_copy(ys_hbm.at[de_v[b]], o_v[b], sg[b])
        go = pltpu.async_copy(ys_hbm.at[do_v[b]], r_v[b], sr[b])
        return ge, go

    gd = {}
    od = {}
    gd[0] = stage(0, 0)
    for j in range(NSUBC):
        b = j % 2
        if j + 1 < NSUBC:
            if j >= 1:
                od[j - 1].wait()
            gd[j + 1] = stage(j + 1, 1 - b)
        gd[j][0].wait()
        gd[j][1].wait()

        def body(t, carry):
            g0 = g_all[j * 2 * STOK + 2 * t, :]
            g1 = g_all[j * 2 * STOK + 2 * t + 1, :]
            for cc in range(Dm // 16):
                sl = pl.ds(cc * 16, 16)
                o_v[b][t, sl] = g0 * o_v[b][t, sl] + g1 * r_v[b][t, sl]
            return carry

        lax.fori_loop(0, STOK, body, 0)
        t0 = pl.multiple_of(wid * TCHUNK + j * STOK, STOK)
        od[j] = pltpu.async_copy(o_v[b], out_hbm.at[pl.ds(t0, STOK)], so[b])
    od[NSUBC - 2].wait()
    od[NSUBC - 1].wait()


_combine = functools.partial(
    pl.kernel,
    _combine_body,
    compiler_params=pltpu.CompilerParams(needs_layout_passes=False),
    out_type=jax.ShapeDtypeStruct((T, Dm), jnp.float32),
    scratch_types=[
        pltpu.VMEM((2 * TCHUNK,), jnp.int32),
        pltpu.VMEM((2 * TCHUNK, 16), jnp.float32),
        [pltpu.VMEM((STOK,), jnp.int32)] * 2,
        [pltpu.VMEM((STOK,), jnp.int32)] * 2,
        [pltpu.VMEM((STOK, Dm), jnp.float32)] * 2,
        [pltpu.VMEM((STOK, Dm), jnp.float32)] * 2,
        [pltpu.SemaphoreType.DMA] * 2,
        [pltpu.SemaphoreType.DMA] * 2,
        [pltpu.SemaphoreType.DMA] * 2,
    ],
)


def kernel(x, W1, b1, W2, b2, Wr, br):
    xf = x.reshape(T, Dm)
    ints, flts, base_o, cumt_o, xp = _router(xf, Wr, br.reshape(1, E))
    eids = ints[:, 0:2].reshape(PAIRS)
    pdest = ints[:, 2:4].reshape(PAIRS)
    gates16 = jnp.broadcast_to(flts.reshape(PAIRS, 1), (PAIRS, 16))
    base16 = jnp.concatenate(
        [base_o.reshape(E), jnp.zeros((16 - E,), jnp.int32)])
    cumt = cumt_o.reshape(E)

    mesh = plsc.VectorSubcoreMesh(core_axis_name="c", subcore_axis_name="s")
    xs, dest = _dispatch(mesh=mesh)(xp, pdest, eids, base16)
    W1bf = W1.astype(jnp.bfloat16)
    ys = _gemm(cumt, xs, W1bf[:, :, :Dm // 2], W1bf[:, :, Dm // 2:],
               b1.reshape(E, 1, H), W2.astype(jnp.bfloat16),
               b2.reshape(E, 1, O))
    out = _combine(mesh=mesh)(ys, dest, gates16)
    return out.reshape(B, N_OBJ, O)
